# baseline (speedup 1.0000x reference)
import functools

import jax
import jax.numpy as jnp
from jax import lax
from jax.experimental import pallas as pl
from jax.experimental.pallas import tpu as pltpu
from jax.experimental.pallas import tpu_sc as plsc

N = 320000
D = 128
K = 1024
NC = 2
NS = 16
NW = NC * NS
LANES = 16
ROWS_W = N // NW
CH = 80
NCHUNK = ROWS_W // CH
NBUF = 5
AUXW = 2 * LANES

_mesh = plsc.VectorSubcoreMesh(core_axis_name="c", subcore_axis_name="s")


@functools.partial(
    pl.kernel,
    out_type=[
        jax.ShapeDtypeStruct((NC, K, D), jnp.float32),
        jax.ShapeDtypeStruct((NC, K, AUXW), jnp.float32),
    ],
    mesh=_mesh,
    compiler_params=pltpu.CompilerParams(use_tc_tiling_on_sc=False),
    scratch_types=(
        [pltpu.VMEM((CH, D), jnp.float32)] * NBUF +
        [pltpu.VMEM((CH, AUXW), jnp.float32)] * NBUF +
        [pltpu.VMEM((CH,), jnp.int32)] * NBUF +
        [pltpu.VMEM_SHARED((K, D), jnp.float32),
         pltpu.VMEM_SHARED((K, AUXW), jnp.float32)] +
        [pltpu.SemaphoreType.DMA] * (4 * NBUF)
    ),
)
def _accum_kernel(var_hbm, lab_hbm, sums_out, aux_out, *refs):
    xb = refs[0:NBUF]
    zb = refs[NBUF:2 * NBUF]
    ib = refs[2 * NBUF:3 * NBUF]
    ssums = refs[3 * NBUF]
    saux = refs[3 * NBUF + 1]
    sems = refs[3 * NBUF + 2:]
    inx = sems[0:NBUF]
    inl = sems[NBUF:2 * NBUF]
    scx = sems[2 * NBUF:3 * NBUF]
    scz = sems[3 * NBUF:4 * NBUF]

    cid = lax.axis_index("c")
    sid = lax.axis_index("s")
    wid = cid * NS + sid
    zeros16 = jnp.zeros((LANES,), jnp.float32)
    x0, z0 = xb[0], zb[0]

    KSLICE = K // NS

    @plsc.parallel_loop(0, KSLICE, unroll=4)
    def _zrow(r):
        for j in range(D // LANES):
            x0[r, pl.ds(j * LANES, LANES)] = zeros16
        for j in range(AUXW // LANES):
            z0[r, pl.ds(j * LANES, LANES)] = zeros16

    pltpu.sync_copy(x0.at[pl.ds(0, KSLICE)],
                    ssums.at[pl.ds(sid * KSLICE, KSLICE)])
    pltpu.sync_copy(z0.at[pl.ds(0, KSLICE)],
                    saux.at[pl.ds(sid * KSLICE, KSLICE)])
    plsc.subcore_barrier()

    lanes = jnp.arange(LANES, dtype=jnp.int32)
    onehot = jnp.where(lanes == 0, 1.0, 0.0).astype(jnp.float32)

    @plsc.parallel_loop(0, CH, unroll=4)
    def _crow(r):
        for z in zb:
            z[r, pl.ds(0, LANES)] = zeros16
            z[r, pl.ds(LANES, LANES)] = onehot

    row0 = wid * ROWS_W
    pltpu.async_copy(var_hbm.at[pl.ds(row0, CH)], xb[0], inx[0])
    pltpu.async_copy(lab_hbm.at[pl.ds(row0, CH)], ib[0], inl[0])

    POUT = NCHUNK // NBUF

    def _group(p, _):
        base_p = pl.multiple_of(row0 + p * NBUF * CH, 8)
        for b in range(NBUF):
            pltpu.make_async_copy(
                var_hbm.at[pl.ds(base_p + b * CH, CH)], xb[b],
                inx[b]).wait()
            pltpu.make_async_copy(
                lab_hbm.at[pl.ds(base_p + b * CH, CH)], ib[b],
                inl[b]).wait()

            xref = xb[b]
            zref = zb[b]

            @plsc.parallel_loop(0, CH, unroll=4)
            def _row(r):
                acc = zeros16
                for j in range(D // LANES):
                    v = xref[r, pl.ds(j * LANES, LANES)]
                    acc = acc + v * v
                zref[r, pl.ds(0, LANES)] = acc

            pltpu.async_copy(xb[b], ssums.at[ib[b]], scx[b], add=True)
            pltpu.async_copy(zb[b], saux.at[ib[b]], scz[b], add=True)

            nb = (b + 1) % NBUF
            nbase = pl.multiple_of(base_p + (b + 1) * CH, 8)

            def _prefetch(first):
                if not first:
                    pltpu.make_async_copy(
                        xb[nb], ssums.at[ib[nb]], scx[nb]).wait()
                    pltpu.make_async_copy(
                        zb[nb], saux.at[ib[nb]], scz[nb]).wait()
                pltpu.async_copy(var_hbm.at[pl.ds(nbase, CH)],
                                 xb[nb], inx[nb])
                pltpu.async_copy(lab_hbm.at[pl.ds(nbase, CH)],
                                 ib[nb], inl[nb])

            if b < NBUF - 1:
                @pl.when(p > 0)
                def _():
                    _prefetch(False)

                @pl.when(p == 0)
                def _():
                    _prefetch(True)
            else:
                @pl.when(p + 1 < POUT)
                def _():
                    _prefetch(False)
        return 0
    lax.fori_loop(0, POUT, _group, 0)

    for b in range(NBUF):
        pltpu.make_async_copy(xb[b], ssums.at[ib[b]], scx[b]).wait()
        pltpu.make_async_copy(zb[b], saux.at[ib[b]], scz[b]).wait()

    plsc.subcore_barrier()

    pltpu.sync_copy(ssums.at[pl.ds(sid * KSLICE, KSLICE)],
                    sums_out.at[cid, pl.ds(sid * KSLICE, KSLICE)])
    pltpu.sync_copy(saux.at[pl.ds(sid * KSLICE, KSLICE)],
                    aux_out.at[cid, pl.ds(sid * KSLICE, KSLICE)])


SCH = K // NS


@functools.partial(
    pl.kernel,
    out_type=jax.ShapeDtypeStruct((LANES,), jnp.float32),
    mesh=_mesh,
    compiler_params=pltpu.CompilerParams(use_tc_tiling_on_sc=False),
    scratch_types=[
        pltpu.VMEM((SCH, D), jnp.float32),
        pltpu.VMEM((SCH, D), jnp.float32),
        pltpu.VMEM((SCH, AUXW), jnp.float32),
        pltpu.VMEM((SCH, AUXW), jnp.float32),
        pltpu.VMEM((D,), jnp.float32),
        pltpu.VMEM((NS, D), jnp.float32),
        pltpu.VMEM((LANES,), jnp.float32),
        pltpu.VMEM_SHARED((NS, D), jnp.float32),
    ],
)
def _loss_kernel(sums_p, aux_p, out_hbm, s0, s1, a0, a1,
                 stg, fold, outbuf, sstage):
    cid = lax.axis_index("c")
    sid = lax.axis_index("s")

    @pl.when(cid == 0)
    def _():
        lanes = jnp.arange(LANES, dtype=jnp.int32)
        zeros16 = jnp.zeros((LANES,), jnp.float32)
        off = sid * SCH
        pltpu.sync_copy(sums_p.at[0, pl.ds(off, SCH)], s0)
        pltpu.sync_copy(sums_p.at[1, pl.ds(off, SCH)], s1)
        pltpu.sync_copy(aux_p.at[0, pl.ds(off, SCH)], a0)
        pltpu.sync_copy(aux_p.at[1, pl.ds(off, SCH)], a1)

        def _seg(r, carry):
            loss_v, nuniq_s = carry
            sacc = zeros16
            for j in range(D // LANES):
                v = (s0[r, pl.ds(j * LANES, LANES)] +
                     s1[r, pl.ds(j * LANES, LANES)])
                sacc = sacc + v * v
            qv = a0[r, pl.ds(0, LANES)] + a1[r, pl.ds(0, LANES)]
            cv = (a0[r, pl.ds(LANES, LANES)] +
                  a1[r, pl.ds(LANES, LANES)])
            c = cv[0]

            segid = off + r
            present = c > 0.0
            valid = present & (segid != 0)
            safec = jnp.where(present, c, 1.0)
            contrib = jnp.where(
                valid, (qv - sacc / safec) / (safec * float(D)),
                zeros16)
            nuniq_s = nuniq_s + jnp.where(present, 1.0, 0.0)
            return loss_v + contrib, nuniq_s

        loss_v, nuniq = lax.fori_loop(0, SCH, _seg, (zeros16, 0.0))

        stg[pl.ds(0, LANES)] = loss_v
        stg[pl.ds(LANES, LANES)] = jnp.where(lanes == 0, nuniq, 0.0)
        for j in range(2, D // LANES):
            stg[pl.ds(j * LANES, LANES)] = zeros16
        pltpu.sync_copy(stg, sstage.at[sid])
        plsc.subcore_barrier()

        @pl.when(sid == 0)
        def _():
            pltpu.sync_copy(sstage, fold)
            lv = zeros16
            nv = zeros16
            for t in range(NS):
                lv = lv + fold[t, pl.ds(0, LANES)]
                nv = nv + fold[t, pl.ds(LANES, LANES)]
            loss = lv[0]
            for i in range(1, LANES):
                loss = loss + lv[i]
            nuniq_tot = nv[0]
            loss = jnp.where(nuniq_tot == 1.0, 0.0, loss)
            outbuf[pl.ds(0, LANES)] = jnp.where(lanes == 0, loss, 0.0)
            pltpu.sync_copy(outbuf, out_hbm)


def kernel(variances, ins_labels):
    sums_p, aux_p = _accum_kernel(variances, ins_labels.astype(jnp.int32))
    out = _loss_kernel(sums_p, aux_p)
    return out[0]


# speedup vs baseline: 1.0775x; 1.0775x over previous
import functools

import jax
import jax.numpy as jnp
from jax import lax
from jax.experimental import pallas as pl
from jax.experimental.pallas import tpu as pltpu
from jax.experimental.pallas import tpu_sc as plsc

N = 320000
D = 128
K = 1024
NC = 2
NS = 16
NW = NC * NS
LANES = 16
ROWS_W = N // NW
CH = 200
NCHUNK = ROWS_W // CH
NBUF = 3
AUXW = 2 * LANES

_mesh = plsc.VectorSubcoreMesh(core_axis_name="c", subcore_axis_name="s")


@functools.partial(
    pl.kernel,
    out_type=[
        jax.ShapeDtypeStruct((NC, K, D), jnp.float32),
        jax.ShapeDtypeStruct((NC, K, AUXW), jnp.float32),
    ],
    mesh=_mesh,
    compiler_params=pltpu.CompilerParams(use_tc_tiling_on_sc=False),
    scratch_types=(
        [pltpu.VMEM((CH, D), jnp.float32)] * NBUF +
        [pltpu.VMEM((CH, AUXW), jnp.float32)] * NBUF +
        [pltpu.VMEM((CH,), jnp.int32)] * NBUF +
        [pltpu.VMEM_SHARED((K, D), jnp.float32),
         pltpu.VMEM_SHARED((K, AUXW), jnp.float32)] +
        [pltpu.SemaphoreType.DMA] * (4 * NBUF)
    ),
)
def _accum_kernel(var_hbm, lab_hbm, sums_out, aux_out, *refs):
    xb = refs[0:NBUF]
    zb = refs[NBUF:2 * NBUF]
    ib = refs[2 * NBUF:3 * NBUF]
    ssums = refs[3 * NBUF]
    saux = refs[3 * NBUF + 1]
    sems = refs[3 * NBUF + 2:]
    inx = sems[0:NBUF]
    inl = sems[NBUF:2 * NBUF]
    scx = sems[2 * NBUF:3 * NBUF]
    scz = sems[3 * NBUF:4 * NBUF]

    cid = lax.axis_index("c")
    sid = lax.axis_index("s")
    wid = cid * NS + sid
    zeros16 = jnp.zeros((LANES,), jnp.float32)
    x0, z0 = xb[0], zb[0]

    KSLICE = K // NS

    @plsc.parallel_loop(0, KSLICE, unroll=4)
    def _zrow(r):
        for j in range(D // LANES):
            x0[r, pl.ds(j * LANES, LANES)] = zeros16
        for j in range(AUXW // LANES):
            z0[r, pl.ds(j * LANES, LANES)] = zeros16

    pltpu.sync_copy(x0.at[pl.ds(0, KSLICE)],
                    ssums.at[pl.ds(sid * KSLICE, KSLICE)])
    pltpu.sync_copy(z0.at[pl.ds(0, KSLICE)],
                    saux.at[pl.ds(sid * KSLICE, KSLICE)])
    plsc.subcore_barrier()

    lanes = jnp.arange(LANES, dtype=jnp.int32)
    onehot = jnp.where(lanes == 0, 1.0, 0.0).astype(jnp.float32)

    @plsc.parallel_loop(0, CH, unroll=4)
    def _crow(r):
        for z in zb:
            z[r, pl.ds(0, LANES)] = zeros16
            z[r, pl.ds(LANES, LANES)] = onehot

    row0 = wid * ROWS_W
    for b in (0, 1):
        pltpu.async_copy(var_hbm.at[pl.ds(row0 + b * CH, CH)],
                         xb[b], inx[b])
        pltpu.async_copy(lab_hbm.at[pl.ds(row0 + b * CH, CH)],
                         ib[b], inl[b])

    POUT = -(-NCHUNK // NBUF)

    def _group(p, _):
        base_p = pl.multiple_of(row0 + p * NBUF * CH, 8)

        def _slot(b):
            pltpu.make_async_copy(
                var_hbm.at[pl.ds(base_p + b * CH, CH)], xb[b],
                inx[b]).wait()
            pltpu.make_async_copy(
                lab_hbm.at[pl.ds(base_p + b * CH, CH)], ib[b],
                inl[b]).wait()

            xref = xb[b]
            zref = zb[b]

            @plsc.parallel_loop(0, CH, unroll=4)
            def _row(r):
                acc = zeros16
                for j in range(D // LANES):
                    v = xref[r, pl.ds(j * LANES, LANES)]
                    acc = acc + v * v
                zref[r, pl.ds(0, LANES)] = acc

            pltpu.async_copy(xb[b], ssums.at[ib[b]], scx[b], add=True)
            pltpu.async_copy(zb[b], saux.at[ib[b]], scz[b], add=True)

            nb = (b + 2) % NBUF
            nbase = pl.multiple_of(base_p + (b + 2) * CH, 8)

            def _prefetch(first):
                if not first:
                    pltpu.make_async_copy(
                        xb[nb], ssums.at[ib[nb]], scx[nb]).wait()
                    pltpu.make_async_copy(
                        zb[nb], saux.at[ib[nb]], scz[nb]).wait()
                pltpu.async_copy(var_hbm.at[pl.ds(nbase, CH)],
                                 xb[nb], inx[nb])
                pltpu.async_copy(lab_hbm.at[pl.ds(nbase, CH)],
                                 ib[nb], inl[nb])

            if b == 0:
                @pl.when(p == 0)
                def _():
                    _prefetch(True)

                @pl.when((p > 0) & (p < POUT - 1))
                def _():
                    _prefetch(False)
            else:
                @pl.when(p < POUT - 1)
                def _():
                    _prefetch(False)

        for b in range(NBUF):
            if b < NCHUNK - (POUT - 1) * NBUF:
                _slot(b)
            else:
                @pl.when(p < POUT - 1)
                def _():
                    _slot(b)
        return 0
    lax.fori_loop(0, POUT, _group, 0)

    for b in range(NBUF):
        pltpu.make_async_copy(xb[b], ssums.at[ib[b]], scx[b]).wait()
        pltpu.make_async_copy(zb[b], saux.at[ib[b]], scz[b]).wait()

    plsc.subcore_barrier()

    pltpu.sync_copy(ssums.at[pl.ds(sid * KSLICE, KSLICE)],
                    sums_out.at[cid, pl.ds(sid * KSLICE, KSLICE)])
    pltpu.sync_copy(saux.at[pl.ds(sid * KSLICE, KSLICE)],
                    aux_out.at[cid, pl.ds(sid * KSLICE, KSLICE)])


SCH = K // NS


@functools.partial(
    pl.kernel,
    out_type=jax.ShapeDtypeStruct((LANES,), jnp.float32),
    mesh=_mesh,
    compiler_params=pltpu.CompilerParams(use_tc_tiling_on_sc=False),
    scratch_types=[
        pltpu.VMEM((SCH, D), jnp.float32),
        pltpu.VMEM((SCH, D), jnp.float32),
        pltpu.VMEM((SCH, AUXW), jnp.float32),
        pltpu.VMEM((SCH, AUXW), jnp.float32),
        pltpu.VMEM((D,), jnp.float32),
        pltpu.VMEM((NS, D), jnp.float32),
        pltpu.VMEM((LANES,), jnp.float32),
        pltpu.VMEM_SHARED((NS, D), jnp.float32),
    ],
)
def _loss_kernel(sums_p, aux_p, out_hbm, s0, s1, a0, a1,
                 stg, fold, outbuf, sstage):
    cid = lax.axis_index("c")
    sid = lax.axis_index("s")

    @pl.when(cid == 0)
    def _():
        lanes = jnp.arange(LANES, dtype=jnp.int32)
        zeros16 = jnp.zeros((LANES,), jnp.float32)
        off = sid * SCH
        pltpu.sync_copy(sums_p.at[0, pl.ds(off, SCH)], s0)
        pltpu.sync_copy(sums_p.at[1, pl.ds(off, SCH)], s1)
        pltpu.sync_copy(aux_p.at[0, pl.ds(off, SCH)], a0)
        pltpu.sync_copy(aux_p.at[1, pl.ds(off, SCH)], a1)

        def _seg(r, carry):
            loss_v, nuniq_s = carry
            sacc = zeros16
            for j in range(D // LANES):
                v = (s0[r, pl.ds(j * LANES, LANES)] +
                     s1[r, pl.ds(j * LANES, LANES)])
                sacc = sacc + v * v
            qv = a0[r, pl.ds(0, LANES)] + a1[r, pl.ds(0, LANES)]
            cv = (a0[r, pl.ds(LANES, LANES)] +
                  a1[r, pl.ds(LANES, LANES)])
            c = cv[0]

            segid = off + r
            present = c > 0.0
            valid = present & (segid != 0)
            safec = jnp.where(present, c, 1.0)
            contrib = jnp.where(
                valid, (qv - sacc / safec) / (safec * float(D)),
                zeros16)
            nuniq_s = nuniq_s + jnp.where(present, 1.0, 0.0)
            return loss_v + contrib, nuniq_s

        loss_v, nuniq = lax.fori_loop(0, SCH, _seg, (zeros16, 0.0))

        stg[pl.ds(0, LANES)] = loss_v
        stg[pl.ds(LANES, LANES)] = jnp.where(lanes == 0, nuniq, 0.0)
        for j in range(2, D // LANES):
            stg[pl.ds(j * LANES, LANES)] = zeros16
        pltpu.sync_copy(stg, sstage.at[sid])
        plsc.subcore_barrier()

        @pl.when(sid == 0)
        def _():
            pltpu.sync_copy(sstage, fold)
            lv = zeros16
            nv = zeros16
            for t in range(NS):
                lv = lv + fold[t, pl.ds(0, LANES)]
                nv = nv + fold[t, pl.ds(LANES, LANES)]
            loss = lv[0]
            for i in range(1, LANES):
                loss = loss + lv[i]
            nuniq_tot = nv[0]
            loss = jnp.where(nuniq_tot == 1.0, 0.0, loss)
            outbuf[pl.ds(0, LANES)] = jnp.where(lanes == 0, loss, 0.0)
            pltpu.sync_copy(outbuf, out_hbm)


def kernel(variances, ins_labels):
    sums_p, aux_p = _accum_kernel(variances, ins_labels.astype(jnp.int32))
    out = _loss_kernel(sums_p, aux_p)
    return out[0]


# speedup vs baseline: 1.0848x; 1.0067x over previous
import functools

import jax
import jax.numpy as jnp
from jax import lax
from jax.experimental import pallas as pl
from jax.experimental.pallas import tpu as pltpu
from jax.experimental.pallas import tpu_sc as plsc

N = 320000
D = 128
K = 1024
NC = 2
NS = 16
NW = NC * NS
LANES = 16
ROWS_W = N // NW
CH = 400
NCHUNK = ROWS_W // CH
NBUF = 2
AUXW = 2 * LANES

_mesh = plsc.VectorSubcoreMesh(core_axis_name="c", subcore_axis_name="s")


@functools.partial(
    pl.kernel,
    out_type=[
        jax.ShapeDtypeStruct((NC, K, D), jnp.float32),
        jax.ShapeDtypeStruct((NC, K, AUXW), jnp.float32),
    ],
    mesh=_mesh,
    compiler_params=pltpu.CompilerParams(use_tc_tiling_on_sc=False),
    scratch_types=(
        [pltpu.VMEM((CH, D), jnp.float32)] * NBUF +
        [pltpu.VMEM((CH, AUXW), jnp.float32)] +
        [pltpu.VMEM((CH,), jnp.int32)] * NBUF +
        [pltpu.VMEM_SHARED((K, D), jnp.float32),
         pltpu.VMEM_SHARED((K, AUXW), jnp.float32)] +
        [pltpu.SemaphoreType.DMA] * (3 * NBUF + 1)
    ),
)
def _accum_kernel(var_hbm, lab_hbm, sums_out, aux_out, *refs):
    xb = refs[0:NBUF]
    zbuf = refs[NBUF]
    ib = refs[NBUF + 1:2 * NBUF + 1]
    ssums = refs[2 * NBUF + 1]
    saux = refs[2 * NBUF + 2]
    sems = refs[2 * NBUF + 3:]
    inx = sems[0:NBUF]
    inl = sems[NBUF:2 * NBUF]
    scx = sems[2 * NBUF:3 * NBUF]
    scz = sems[3 * NBUF]

    cid = lax.axis_index("c")
    sid = lax.axis_index("s")
    wid = cid * NS + sid
    zeros16 = jnp.zeros((LANES,), jnp.float32)
    x0, z0 = xb[0], zbuf

    KSLICE = K // NS

    @plsc.parallel_loop(0, KSLICE, unroll=4)
    def _zrow(r):
        for j in range(D // LANES):
            x0[r, pl.ds(j * LANES, LANES)] = zeros16
        for j in range(AUXW // LANES):
            z0[r, pl.ds(j * LANES, LANES)] = zeros16

    pltpu.sync_copy(x0.at[pl.ds(0, KSLICE)],
                    ssums.at[pl.ds(sid * KSLICE, KSLICE)])
    pltpu.sync_copy(z0.at[pl.ds(0, KSLICE)],
                    saux.at[pl.ds(sid * KSLICE, KSLICE)])
    plsc.subcore_barrier()

    lanes = jnp.arange(LANES, dtype=jnp.int32)
    onehot = jnp.where(lanes == 0, 1.0, 0.0).astype(jnp.float32)

    @plsc.parallel_loop(0, CH, unroll=4)
    def _crow(r):
        zbuf[r, pl.ds(0, LANES)] = zeros16
        zbuf[r, pl.ds(LANES, LANES)] = onehot

    row0 = wid * ROWS_W
    pltpu.async_copy(var_hbm.at[pl.ds(row0, CH)], xb[0], inx[0])
    pltpu.async_copy(lab_hbm.at[pl.ds(row0, CH)], ib[0], inl[0])

    POUT = -(-NCHUNK // NBUF)

    def _group(p, _):
        base_p = pl.multiple_of(row0 + p * NBUF * CH, 8)

        def _slot(b):
            pltpu.make_async_copy(
                var_hbm.at[pl.ds(base_p + b * CH, CH)], xb[b],
                inx[b]).wait()
            pltpu.make_async_copy(
                lab_hbm.at[pl.ds(base_p + b * CH, CH)], ib[b],
                inl[b]).wait()

            def _zwait():
                pltpu.make_async_copy(zbuf, saux.at[ib[b]], scz).wait()
            if b == 0:
                @pl.when(p > 0)
                def _():
                    _zwait()
            else:
                _zwait()

            xref = xb[b]

            @plsc.parallel_loop(0, CH, unroll=4)
            def _row(r):
                acc = zeros16
                for j in range(D // LANES):
                    v = xref[r, pl.ds(j * LANES, LANES)]
                    acc = acc + v * v
                zbuf[r, pl.ds(0, LANES)] = acc

            pltpu.async_copy(xb[b], ssums.at[ib[b]], scx[b], add=True)
            pltpu.async_copy(zbuf, saux.at[ib[b]], scz, add=True)

            nb = (b + 1) % NBUF
            nbase = pl.multiple_of(base_p + (b + 1) * CH, 8)

            def _prefetch(first):
                if not first:
                    pltpu.make_async_copy(
                        xb[nb], ssums.at[ib[nb]], scx[nb]).wait()
                pltpu.async_copy(var_hbm.at[pl.ds(nbase, CH)],
                                 xb[nb], inx[nb])
                pltpu.async_copy(lab_hbm.at[pl.ds(nbase, CH)],
                                 ib[nb], inl[nb])

            if b == 0:
                @pl.when(p == 0)
                def _():
                    _prefetch(True)

                @pl.when((p > 0) & (p < POUT - 1))
                def _():
                    _prefetch(False)
            else:
                @pl.when(p < POUT - 1)
                def _():
                    _prefetch(False)

        for b in range(NBUF):
            if b < NCHUNK - (POUT - 1) * NBUF:
                _slot(b)
            else:
                @pl.when(p < POUT - 1)
                def _():
                    _slot(b)
        return 0
    lax.fori_loop(0, POUT, _group, 0)

    for b in range(NBUF):
        pltpu.make_async_copy(xb[b], ssums.at[ib[b]], scx[b]).wait()
    pltpu.make_async_copy(zbuf, saux.at[ib[0]], scz).wait()

    plsc.subcore_barrier()

    pltpu.sync_copy(ssums.at[pl.ds(sid * KSLICE, KSLICE)],
                    sums_out.at[cid, pl.ds(sid * KSLICE, KSLICE)])
    pltpu.sync_copy(saux.at[pl.ds(sid * KSLICE, KSLICE)],
                    aux_out.at[cid, pl.ds(sid * KSLICE, KSLICE)])


SCH = K // NS


@functools.partial(
    pl.kernel,
    out_type=jax.ShapeDtypeStruct((LANES,), jnp.float32),
    mesh=_mesh,
    compiler_params=pltpu.CompilerParams(use_tc_tiling_on_sc=False),
    scratch_types=[
        pltpu.VMEM((SCH, D), jnp.float32),
        pltpu.VMEM((SCH, D), jnp.float32),
        pltpu.VMEM((SCH, AUXW), jnp.float32),
        pltpu.VMEM((SCH, AUXW), jnp.float32),
        pltpu.VMEM((D,), jnp.float32),
        pltpu.VMEM((NS, D), jnp.float32),
        pltpu.VMEM((LANES,), jnp.float32),
        pltpu.VMEM_SHARED((NS, D), jnp.float32),
    ],
)
def _loss_kernel(sums_p, aux_p, out_hbm, s0, s1, a0, a1,
                 stg, fold, outbuf, sstage):
    cid = lax.axis_index("c")
    sid = lax.axis_index("s")

    @pl.when(cid == 0)
    def _():
        lanes = jnp.arange(LANES, dtype=jnp.int32)
        zeros16 = jnp.zeros((LANES,), jnp.float32)
        off = sid * SCH
        pltpu.sync_copy(sums_p.at[0, pl.ds(off, SCH)], s0)
        pltpu.sync_copy(sums_p.at[1, pl.ds(off, SCH)], s1)
        pltpu.sync_copy(aux_p.at[0, pl.ds(off, SCH)], a0)
        pltpu.sync_copy(aux_p.at[1, pl.ds(off, SCH)], a1)

        def _seg(r, carry):
            loss_v, nuniq_s = carry
            sacc = zeros16
            for j in range(D // LANES):
                v = (s0[r, pl.ds(j * LANES, LANES)] +
                     s1[r, pl.ds(j * LANES, LANES)])
                sacc = sacc + v * v
            qv = a0[r, pl.ds(0, LANES)] + a1[r, pl.ds(0, LANES)]
            cv = (a0[r, pl.ds(LANES, LANES)] +
                  a1[r, pl.ds(LANES, LANES)])
            c = cv[0]

            segid = off + r
            present = c > 0.0
            valid = present & (segid != 0)
            safec = jnp.where(present, c, 1.0)
            contrib = jnp.where(
                valid, (qv - sacc / safec) / (safec * float(D)),
                zeros16)
            nuniq_s = nuniq_s + jnp.where(present, 1.0, 0.0)
            return loss_v + contrib, nuniq_s

        loss_v, nuniq = lax.fori_loop(0, SCH, _seg, (zeros16, 0.0))

        stg[pl.ds(0, LANES)] = loss_v
        stg[pl.ds(LANES, LANES)] = jnp.where(lanes == 0, nuniq, 0.0)
        for j in range(2, D // LANES):
            stg[pl.ds(j * LANES, LANES)] = zeros16
        pltpu.sync_copy(stg, sstage.at[sid])
        plsc.subcore_barrier()

        @pl.when(sid == 0)
        def _():
            pltpu.sync_copy(sstage, fold)
            lv = zeros16
            nv = zeros16
            for t in range(NS):
                lv = lv + fold[t, pl.ds(0, LANES)]
                nv = nv + fold[t, pl.ds(LANES, LANES)]
            loss = lv[0]
            for i in range(1, LANES):
                loss = loss + lv[i]
            nuniq_tot = nv[0]
            loss = jnp.where(nuniq_tot == 1.0, 0.0, loss)
            outbuf[pl.ds(0, LANES)] = jnp.where(lanes == 0, loss, 0.0)
            pltpu.sync_copy(outbuf, out_hbm)


def kernel(variances, ins_labels):
    sums_p, aux_p = _accum_kernel(variances, ins_labels.astype(jnp.int32))
    out = _loss_kernel(sums_p, aux_p)
    return out[0]
